# Initial kernel scaffold; baseline (speedup 1.0000x reference)
import jax
import jax.numpy as jnp
from jax.experimental import pallas as pl


def kernel(x1, x2, train_set, train_batch):
    raise NotImplementedError("write your pallas kernel here")


# speedup vs baseline: 1.1712x; 1.1712x over previous
import functools

import jax
import jax.numpy as jnp
from jax import lax
from jax.experimental import pallas as pl
from jax.experimental.pallas import tpu as pltpu
from jax.experimental.pallas import tpu_sc as plsc

_GAMMA = 3.0
_N = 100000
_D = 128
_B = 4096
_K = 25
_NC = 2
_NS = 16
_NW = _NC * _NS
_RPW = _B // _NW
_NCHUNK = _RPW // 16
_CPD = _D // 16


def _row_l1_partial(a_ref, b_ref, r):
    p = jnp.abs(a_ref[r, pl.ds(0, 16)] - b_ref[r, pl.ds(0, 16)])
    for c in range(1, _CPD):
        p = p + jnp.abs(a_ref[r, pl.ds(c * 16, 16)] - b_ref[r, pl.ds(c * 16, 16)])
    return p


def _row_l1(a_ref, b_ref, r):
    return jnp.sum(_row_l1_partial(a_ref, b_ref, r))


def _make_sc_main():
    mesh = plsc.VectorSubcoreMesh(core_axis_name="c", subcore_axis_name="s")

    @functools.partial(
        pl.kernel,
        out_type=jax.ShapeDtypeStruct((_NW, 16), jnp.float32),
        mesh=mesh,
        compiler_params=pltpu.CompilerParams(needs_layout_passes=False),
        scratch_types=[
            pltpu.VMEM((_RPW,), jnp.int32),
            pltpu.VMEM((_RPW, _D), jnp.float32),
            pltpu.VMEM((_RPW, _D), jnp.float32),
            pltpu.VMEM((_RPW, _D), jnp.float32),
            pltpu.SMEM((_RPW,), jnp.float32),
            pltpu.VMEM((16,), jnp.float32),
            pltpu.SemaphoreType.DMA,
        ],
    )
    def sc_main(x1_hbm, x2_hbm, ts0_hbm, ts1_hbm, tb_hbm, out_hbm,
                idx_v, a1_v, a2_v, neg_v, dis_v, ovec_v, sem):
        wid = lax.axis_index("s") * _NC + lax.axis_index("c")
        base = wid * _RPW

        pltpu.sync_copy(ts0_hbm.at[pl.ds(base, _RPW)], idx_v)
        pltpu.async_copy(x1_hbm.at[idx_v], a1_v, sem).wait()
        pltpu.sync_copy(ts1_hbm.at[pl.ds(base, _RPW)], idx_v)
        pltpu.async_copy(x2_hbm.at[idx_v], a2_v, sem).wait()

        def dis_body(r, _):
            dis_v[r] = _row_l1(a1_v, a2_v, r)
            return 0

        lax.fori_loop(0, _RPW, dis_body, 0)

        acc = jnp.float32(0.0)
        for g in range(4):
            a_ref = a1_v if g < 2 else a2_v
            tab_hbm = (x1_hbm, x2_hbm, x2_hbm, x1_hbm)[g]

            def blk_body(k, acc, g=g, a_ref=a_ref, tab_hbm=tab_hbm):
                pltpu.sync_copy(tb_hbm.at[g, k, pl.ds(base, _RPW)], idx_v)
                pltpu.async_copy(tab_hbm.at[idx_v], neg_v, sem).wait()

                def row_acc(r, acc):
                    dneg = _row_l1(a_ref, neg_v, r)
                    return acc + jnp.maximum(_GAMMA + dis_v[r] - dneg, 0.0)

                return lax.fori_loop(0, _RPW, row_acc, acc)

            acc = lax.fori_loop(0, _K, blk_body, acc)

        ovec_v[...] = jnp.full((16,), acc, jnp.float32)
        pltpu.sync_copy(ovec_v, out_hbm.at[wid])

    return sc_main


_sc_main = _make_sc_main()


def _reduce_body(p_ref, o_ref):
    total = jnp.sum(p_ref[...]) * (1.0 / (4 * _K * _B * 16))
    o_ref[...] = jnp.reshape(total, (1, 1))


def kernel(x1, x2, train_set, train_batch):
    ts = train_set.astype(jnp.int32)
    tb = train_batch.astype(jnp.int32).reshape(4, _K, _B)
    ts0 = ts[:, 0]
    ts1 = ts[:, 1]
    partials = _sc_main(x1, x2, ts0, ts1, tb)
    loss2d = pl.pallas_call(
        _reduce_body,
        out_shape=jax.ShapeDtypeStruct((1, 1), jnp.float32),
    )(partials)
    return loss2d[0, 0]


# speedup vs baseline: 2.2998x; 1.9636x over previous
import functools

import jax
import jax.numpy as jnp
from jax import lax
from jax.experimental import pallas as pl
from jax.experimental.pallas import tpu as pltpu
from jax.experimental.pallas import tpu_sc as plsc

_GAMMA = 3.0
_D = 128
_B = 4096
_K = 25
_NC = 2
_NS = 16
_NW = _NC * _NS
_RPW = _B // _NW
_CPD = _D // 16
_IDXLEN = 2 * _RPW + 4 * _K * _RPW


def _row_l1_partial(a_ref, b_ref, r):
    p = jnp.abs(a_ref[r, pl.ds(0, 16)] - b_ref[r, pl.ds(0, 16)])
    for c in range(1, _CPD):
        p = p + jnp.abs(a_ref[r, pl.ds(c * 16, 16)] - b_ref[r, pl.ds(c * 16, 16)])
    return p


def _row_l1(a_ref, b_ref, r):
    return jnp.sum(_row_l1_partial(a_ref, b_ref, r))


def _make_sc_main():
    mesh = plsc.VectorSubcoreMesh(core_axis_name="c", subcore_axis_name="s")

    @functools.partial(
        pl.kernel,
        out_type=jax.ShapeDtypeStruct((_NW, 16), jnp.float32),
        mesh=mesh,
        compiler_params=pltpu.CompilerParams(needs_layout_passes=False),
        scratch_types=[
            pltpu.VMEM((_IDXLEN,), jnp.int32),
            pltpu.VMEM((_RPW, _D), jnp.float32),
            pltpu.VMEM((_RPW, _D), jnp.float32),
            pltpu.VMEM((_RPW, _D), jnp.float32),
            pltpu.VMEM((_RPW, _D), jnp.float32),
            pltpu.SMEM((_RPW,), jnp.float32),
            pltpu.VMEM((16,), jnp.float32),
            pltpu.SemaphoreType.DMA,
            pltpu.SemaphoreType.DMA,
        ],
    )
    def sc_main(x1_hbm, x2_hbm, idx_hbm, out_hbm,
                idx_v, a1_v, a2_v, nb0_v, nb1_v, dis_s, ovec_v, sem0, sem1):
        wid = lax.axis_index("s") * _NC + lax.axis_index("c")

        pltpu.sync_copy(idx_hbm.at[wid], idx_v)

        c1 = pltpu.async_copy(x1_hbm.at[idx_v.at[pl.ds(0, _RPW)]], a1_v, sem0)
        c2 = pltpu.async_copy(x2_hbm.at[idx_v.at[pl.ds(_RPW, _RPW)]], a2_v, sem1)
        c1.wait()
        c2.wait()

        def dis_body(r, _):
            dis_s[r] = _row_l1(a1_v, a2_v, r)
            return 0

        lax.fori_loop(0, _RPW, dis_body, 0, unroll=2)

        def neg_idx(j):
            return idx_v.at[pl.ds(2 * _RPW + j * _RPW, _RPW)]

        bufs = ((nb0_v, sem0), (nb1_v, sem1))
        acc = jnp.float32(0.0)
        for g in range(4):
            a_ref = a1_v if g < 2 else a2_v
            tab_hbm = (x1_hbm, x2_hbm, x2_hbm, x1_hbm)[g]
            jbase = g * _K

            pltpu.async_copy(tab_hbm.at[neg_idx(jbase)], nb0_v, sem0)
            pltpu.async_copy(tab_hbm.at[neg_idx(jbase + 1)], nb1_v, sem1)

            def block(j, nb_v, sem, acc, a_ref=a_ref, tab_hbm=tab_hbm):
                pltpu.make_async_copy(tab_hbm.at[neg_idx(j)], nb_v, sem).wait()

                def row_acc(r, acc):
                    dneg = _row_l1(a_ref, nb_v, r)
                    return acc + jnp.maximum(_GAMMA + dis_s[r] - dneg, 0.0)

                return lax.fori_loop(0, _RPW, row_acc, acc, unroll=4)

            def pair_body(i, acc, a_ref=a_ref, tab_hbm=tab_hbm, jbase=jbase):
                k = i * 2
                for b, (nb_v, sem) in enumerate(bufs):
                    acc = block(jbase + k + b, nb_v, sem, acc)

                    @pl.when(k + b + 2 <= _K - 1)
                    def _(nb_v=nb_v, sem=sem, j2=jbase + k + b + 2,
                          tab_hbm=tab_hbm):
                        pltpu.async_copy(tab_hbm.at[neg_idx(j2)], nb_v, sem)

                return acc

            acc = lax.fori_loop(0, (_K - 1) // 2, pair_body, acc)
            acc = block(jbase + _K - 1, nb0_v, sem0, acc)

        ovec_v[...] = jnp.full((16,), acc, jnp.float32)
        pltpu.sync_copy(ovec_v, out_hbm.at[wid])

    return sc_main


_sc_main = _make_sc_main()


def _reduce_body(p_ref, o_ref):
    total = jnp.sum(p_ref[...]) * (1.0 / (4 * _K * _B * 16))
    o_ref[...] = jnp.reshape(total, (1, 1))


def kernel(x1, x2, train_set, train_batch):
    ts = train_set.astype(jnp.int32)
    tb = train_batch.astype(jnp.int32)
    ts0 = ts[:, 0].reshape(_NW, _RPW)
    ts1 = ts[:, 1].reshape(_NW, _RPW)
    tbw = (tb.reshape(4, _K, _NW, _RPW)
             .transpose(2, 0, 1, 3)
             .reshape(_NW, 4 * _K * _RPW))
    idx_blob = jnp.concatenate([ts0, ts1, tbw], axis=1)
    partials = _sc_main(x1, x2, idx_blob)
    loss2d = pl.pallas_call(
        _reduce_body,
        out_shape=jax.ShapeDtypeStruct((1, 1), jnp.float32),
    )(partials)
    return loss2d[0, 0]


# speedup vs baseline: 2.8889x; 1.2562x over previous
import functools

import jax
import jax.numpy as jnp
from jax import lax
from jax.experimental import pallas as pl
from jax.experimental.pallas import tpu as pltpu
from jax.experimental.pallas import tpu_sc as plsc

_GAMMA = 3.0
_D = 128
_B = 4096
_K = 25
_NC = 2
_NS = 16
_NW = _NC * _NS
_RPW = _B // _NW
_CPD = _D // 16
_IDXLEN = 2 * _RPW + 4 * _K * _RPW


def _row_l1_partial(a_ref, b_ref, r):
    p = jnp.abs(a_ref[r, pl.ds(0, 16)] - b_ref[r, pl.ds(0, 16)])
    for c in range(1, _CPD):
        p = p + jnp.abs(a_ref[r, pl.ds(c * 16, 16)] - b_ref[r, pl.ds(c * 16, 16)])
    return p


def _row_l1(a_ref, b_ref, r):
    return jnp.sum(_row_l1_partial(a_ref, b_ref, r))


def _make_sc_main():
    mesh = plsc.VectorSubcoreMesh(core_axis_name="c", subcore_axis_name="s")

    @functools.partial(
        pl.kernel,
        out_type=jax.ShapeDtypeStruct((_NW, 16), jnp.float32),
        mesh=mesh,
        compiler_params=pltpu.CompilerParams(needs_layout_passes=False),
        scratch_types=[
            pltpu.VMEM((_IDXLEN,), jnp.int32),
            pltpu.VMEM((_RPW, _D), jnp.float32),
            pltpu.VMEM((_RPW, _D), jnp.float32),
            pltpu.VMEM((_RPW, _D), jnp.float32),
            pltpu.VMEM((_RPW, _D), jnp.float32),
            pltpu.VMEM((_RPW, _D), jnp.float32),
            pltpu.VMEM((_RPW, _D), jnp.float32),
            pltpu.SMEM((_RPW,), jnp.float32),
            pltpu.VMEM((16,), jnp.float32),
            pltpu.SemaphoreType.DMA,
            pltpu.SemaphoreType.DMA,
            pltpu.SemaphoreType.DMA,
            pltpu.SemaphoreType.DMA,
            pltpu.SemaphoreType.DMA,
            pltpu.SemaphoreType.DMA,
        ],
    )
    def sc_main(x1_hbm, x2_hbm, idx_hbm, out_hbm,
                idx_v, a1_v, a2_v, nb0_v, nb1_v, nb2_v, nb3_v, dis_s, ovec_v,
                sema0, sema1, semn0, semn1, semn2, semn3):
        wid = lax.axis_index("s") * _NC + lax.axis_index("c")
        nbufs = (nb0_v, nb1_v, nb2_v, nb3_v)
        sems = (semn0, semn1, semn2, semn3)
        tabs = (x1_hbm, x2_hbm, x2_hbm, x1_hbm)

        pltpu.sync_copy(idx_hbm.at[wid], idx_v)

        def neg_idx(j):
            return idx_v.at[pl.ds(2 * _RPW + j * _RPW, _RPW)]

        c1 = pltpu.async_copy(x1_hbm.at[idx_v.at[pl.ds(0, _RPW)]], a1_v, sema0)
        c2 = pltpu.async_copy(x2_hbm.at[idx_v.at[pl.ds(_RPW, _RPW)]], a2_v, sema1)
        for b in range(4):
            pltpu.async_copy(tabs[0].at[neg_idx(b)], nbufs[b], sems[b])
        c1.wait()
        c2.wait()

        def dis_body(r, _):
            dis_s[r] = _row_l1(a1_v, a2_v, r)
            return 0

        lax.fori_loop(0, _RPW, dis_body, 0, unroll=2)

        acc = jnp.float32(0.0)
        for g in range(4):
            a_ref = a1_v if g < 2 else a2_v
            tab_hbm = tabs[g]
            jbase = g * _K

            def pair_rows(n0_v, n1_v, acc, a_ref=a_ref):
                def row_acc(r, acc):
                    d0 = d1 = None
                    for c in range(_CPD):
                        av = a_ref[r, pl.ds(c * 16, 16)]
                        p0 = jnp.abs(av - n0_v[r, pl.ds(c * 16, 16)])
                        p1 = jnp.abs(av - n1_v[r, pl.ds(c * 16, 16)])
                        d0 = p0 if c == 0 else d0 + p0
                        d1 = p1 if c == 0 else d1 + p1
                    gd = _GAMMA + dis_s[r]
                    return (acc + jnp.maximum(gd - jnp.sum(d0), 0.0)
                            + jnp.maximum(gd - jnp.sum(d1), 0.0))

                return lax.fori_loop(0, _RPW, row_acc, acc, unroll=2)

            def two_pairs(i, acc, a_ref=a_ref, tab_hbm=tab_hbm, jbase=jbase):
                for q in (0, 1):
                    n0 = 4 * i + 2 * q
                    j0 = jbase + n0
                    pltpu.make_async_copy(
                        tab_hbm.at[neg_idx(j0)], nbufs[2 * q], sems[2 * q]
                    ).wait()
                    pltpu.make_async_copy(
                        tab_hbm.at[neg_idx(j0 + 1)], nbufs[2 * q + 1],
                        sems[2 * q + 1]
                    ).wait()
                    acc = pair_rows(nbufs[2 * q], nbufs[2 * q + 1], acc)
                    for d in (0, 1):
                        @pl.when(n0 + 4 + d <= _K - 1)
                        def _(j2=j0 + 4 + d, b=2 * q + d, tab_hbm=tab_hbm):
                            pltpu.async_copy(
                                tab_hbm.at[neg_idx(j2)], nbufs[b], sems[b])
                return acc

            acc = lax.fori_loop(0, 6, two_pairs, acc)

            if g < 3:
                for b in (1, 2, 3):
                    pltpu.async_copy(
                        tabs[g + 1].at[neg_idx((g + 1) * _K + b)],
                        nbufs[b], sems[b])

            pltpu.make_async_copy(
                tab_hbm.at[neg_idx(jbase + _K - 1)], nb0_v, semn0).wait()

            def tail_rows(r, acc, a_ref=a_ref):
                dneg = _row_l1(a_ref, nb0_v, r)
                return acc + jnp.maximum(_GAMMA + dis_s[r] - dneg, 0.0)

            acc = lax.fori_loop(0, _RPW, tail_rows, acc, unroll=4)
            if g < 3:
                pltpu.async_copy(
                    tabs[g + 1].at[neg_idx((g + 1) * _K)], nb0_v, semn0)

        ovec_v[...] = jnp.full((16,), acc, jnp.float32)
        pltpu.sync_copy(ovec_v, out_hbm.at[wid])

    return sc_main


_sc_main = _make_sc_main()


def _reduce_body(p_ref, o_ref):
    total = jnp.sum(p_ref[...]) * (1.0 / (4 * _K * _B * 16))
    o_ref[...] = jnp.reshape(total, (1, 1))


def kernel(x1, x2, train_set, train_batch):
    ts = train_set.astype(jnp.int32)
    tb = train_batch.astype(jnp.int32)
    ts0 = ts[:, 0].reshape(_NW, _RPW)
    ts1 = ts[:, 1].reshape(_NW, _RPW)
    tbw = (tb.reshape(4, _K, _NW, _RPW)
             .transpose(2, 0, 1, 3)
             .reshape(_NW, 4 * _K * _RPW))
    idx_blob = jnp.concatenate([ts0, ts1, tbw], axis=1)
    partials = _sc_main(x1, x2, idx_blob)
    loss2d = pl.pallas_call(
        _reduce_body,
        out_shape=jax.ShapeDtypeStruct((1, 1), jnp.float32),
    )(partials)
    return loss2d[0, 0]
